# Initial kernel scaffold; baseline (speedup 1.0000x reference)
import jax
import jax.numpy as jnp
from jax.experimental import pallas as pl


def kernel(attention_scores, bias_table):
    raise NotImplementedError("write your pallas kernel here")


# speedup vs baseline: 1.0163x; 1.0163x over previous
import math

import jax
import jax.numpy as jnp
from jax.experimental import pallas as pl
from jax.experimental.pallas import tpu as pltpu

_NB = 32
_MD = 128
_BR = 512
_S = 2048


def _bias_block(r, table_ref):
    row = jax.lax.broadcasted_iota(jnp.int32, (_BR, _S), 0) + r * _BR
    col = jax.lax.broadcasted_iota(jnp.int32, (_BR, _S), 1)
    d = col - row
    rb = jnp.where(d > 0, _NB // 2, 0)
    a = jnp.abs(d)
    af = a.astype(jnp.float32)
    rp_if_large = _MD + jnp.log(af / _MD) / math.log(_MD / _NB) * (_NB - _MD)
    rp_if_large = jnp.minimum(rp_if_large, _MD - 1)
    large = rb.astype(jnp.float32) + rp_if_large
    small = (a + rb).astype(jnp.float32)
    out = jnp.where(a < _MD, small, large)
    bucket = jnp.clip(out, 0, _NB - 1).astype(jnp.int32)
    acc = jnp.zeros((_BR, _S), jnp.float32)
    for k in range(_NB):
        acc = jnp.where(bucket == k, table_ref[k, 0], acc)
    return acc


def _add_bias_kernel(x_ref, table_ref, o_ref, bias_ref):
    h = pl.program_id(1)

    @pl.when(h == 0)
    def _():
        bias_ref[...] = _bias_block(pl.program_id(0), table_ref)

    o_ref[...] = x_ref[...] + bias_ref[...]


def kernel(attention_scores, bias_table):
    b, h, s, _ = attention_scores.shape
    x = attention_scores.reshape(b * h, s, s)
    grid = (s // _BR, b * h)
    out = pl.pallas_call(
        _add_bias_kernel,
        grid=grid,
        in_specs=[
            pl.BlockSpec((1, _BR, s), lambda r, hh: (hh, r, 0)),
            pl.BlockSpec((_NB, 1), lambda r, hh: (0, 0)),
        ],
        out_specs=pl.BlockSpec((1, _BR, s), lambda r, hh: (hh, r, 0)),
        out_shape=jax.ShapeDtypeStruct((b * h, s, s), jnp.float32),
        scratch_shapes=[pltpu.VMEM((_BR, s), jnp.float32)],
    )(x, bias_table)
    return out.reshape(b, h, s, s)


# speedup vs baseline: 1.0175x; 1.0012x over previous
import math

import jax
import jax.numpy as jnp
from jax.experimental import pallas as pl
from jax.experimental.pallas import tpu as pltpu

_NB = 32
_MD = 128
_BR = 512
_S = 2048


def _bias_block(r, table_ref):
    row = jax.lax.broadcasted_iota(jnp.int32, (_BR, _S), 0) + r * _BR
    col = jax.lax.broadcasted_iota(jnp.int32, (_BR, _S), 1)
    d = col - row
    rb = jnp.where(d > 0, _NB // 2, 0)
    a = jnp.abs(d)
    af = a.astype(jnp.float32)
    rp_if_large = _MD + jnp.log(af / _MD) / math.log(_MD / _NB) * (_NB - _MD)
    rp_if_large = jnp.minimum(rp_if_large, _MD - 1)
    large = rb.astype(jnp.float32) + rp_if_large
    small = (a + rb).astype(jnp.float32)
    out = jnp.where(a < _MD, small, large)
    bucket = jnp.clip(out, 0, _NB - 1).astype(jnp.int32)
    acc = jnp.zeros((_BR, _S), jnp.float32)
    for k in range(_NB):
        acc = jnp.where(bucket == k, table_ref[k, 0], acc)
    return acc


def _add_bias_kernel(x_ref, table_ref, o_ref, bias_ref):
    h = pl.program_id(1)

    @pl.when(h == 0)
    def _():
        bias_ref[...] = _bias_block(pl.program_id(0), table_ref)

    o_ref[...] = x_ref[...] + bias_ref[...]


def kernel(attention_scores, bias_table):
    b, h, s, _ = attention_scores.shape
    x = attention_scores.reshape(b * h, s, s)
    grid = (s // _BR, b * h)
    out = pl.pallas_call(
        _add_bias_kernel,
        grid=grid,
        in_specs=[
            pl.BlockSpec((1, _BR, s), lambda r, hh: (hh, r, 0)),
            pl.BlockSpec((_NB, 1), lambda r, hh: (0, 0)),
        ],
        out_specs=pl.BlockSpec((1, _BR, s), lambda r, hh: (hh, r, 0)),
        out_shape=jax.ShapeDtypeStruct((b * h, s, s), jnp.float32),
        scratch_shapes=[pltpu.VMEM((_BR, s), jnp.float32)],
        compiler_params=pltpu.CompilerParams(
            dimension_semantics=("parallel", "arbitrary")
        ),
    )(x, bias_table)
    return out.reshape(b, h, s, s)
